# baseline (speedup 1.0000x reference)
import functools

import jax
import jax.numpy as jnp
from jax import lax
from jax.experimental import pallas as pl
from jax.experimental.pallas import tpu as pltpu
from jax.experimental.pallas import tpu_sc as plsc

_CU = 609.5593
_CV = 172.854
_FU = 721.5377
_FV = 721.5377
_P03 = 44.85728
_P13 = 0.2163791
_P23 = 0.002745884
_BIG = 1e10

_H = 64
_W = 64
_N = _H * _W
_L = 16
_NC = 2
_NS = 16
_NW = _NC * _NS
_ROWS = _N // _NW
_NJ = _N // _L
_Q = 8
_UNROLL = 2


def _prep_body(t_ref, p_ref, x1, y1, z1, s1, x2, y2, z2, s2, mf):
    t = t_ref[...]
    p = p_ref[...]
    px = lax.broadcasted_iota(jnp.int32, (_H, _W), 1).astype(jnp.float32)
    py = lax.broadcasted_iota(jnp.int32, (_H, _W), 0).astype(jnp.float32)
    m = t > 0.0
    pen = jnp.where(m, 0.0, _BIG)

    def xyz(d):
        x = (px * (d + _P23) - (_CU * d + _P03)) / _FU
        y = (py * (d + _P23) - (_CV * d + _P13)) / _FV
        return x, y, d

    def rb(v):
        return v.astype(jnp.bfloat16).astype(jnp.float32)

    a, b, c = xyz(t)
    x1[...] = rb(a)
    y1[...] = rb(b)
    z1[...] = rb(c)
    s1[...] = a * a + b * b + c * c + pen
    a, b, c = xyz(p)
    x2[...] = rb(a)
    y2[...] = rb(b)
    z2[...] = rb(c)
    s2[...] = a * a + b * b + c * c + pen
    mf[...] = m.astype(jnp.float32)


_prep = pl.pallas_call(
    _prep_body,
    out_shape=[jax.ShapeDtypeStruct((_H, _W), jnp.float32)] * 9,
)


_DNUMS = lax.GatherDimensionNumbers(
    offset_dims=(), collapsed_slice_dims=(0,), start_index_map=(0,))


def _cd_impl(wid, x1h, y1h, z1h, s1h, x2h, y2h, z2h, s2h, mfh,
             o12h, ocmh,
             kx, ky, kz, ks, qx, qy, qz, qs, qm, vcm, vout, sem):
    base = wid * _ROWS

    copies = [
        pltpu.make_async_copy(x2h, kx, sem),
        pltpu.make_async_copy(y2h, ky, sem),
        pltpu.make_async_copy(z2h, kz, sem),
        pltpu.make_async_copy(s2h, ks, sem),
        pltpu.make_async_copy(x1h.at[pl.ds(base, _ROWS)], qx, sem),
        pltpu.make_async_copy(y1h.at[pl.ds(base, _ROWS)], qy, sem),
        pltpu.make_async_copy(z1h.at[pl.ds(base, _ROWS)], qz, sem),
        pltpu.make_async_copy(s1h.at[pl.ds(base, _ROWS)], qs, sem),
        pltpu.make_async_copy(mfh.at[pl.ds(base, _ROWS)], qm, sem),
    ]
    for c in copies:
        c.start()
    for c in copies:
        c.wait()

    def make_qb_body(first):
      def qb_body(qb, acc):
        q0 = qb * _Q
        c0 = (q0 // _L) * _L
        cqx = qx[pl.ds(c0, _L)]
        cqy = qy[pl.ds(c0, _L)]
        cqz = qz[pl.ds(c0, _L)]
        cqs = qs[pl.ds(c0, _L)]
        cqm = qm[pl.ds(c0, _L)]

        def splat(chunk, u):
            lanes = jnp.full((_L,), q0 - c0 + u, jnp.int32)
            return lax.gather(chunk, lanes[:, None], _DNUMS,
                              slice_sizes=(1,),
                              mode=lax.GatherScatterMode.PROMISE_IN_BOUNDS)

        ax = [splat(cqx, u) * -2.0 for u in range(_Q)]
        ay = [splat(cqy, u) * -2.0 for u in range(_Q)]
        az = [splat(cqz, u) * -2.0 for u in range(_Q)]
        sv = [splat(cqs, u) for u in range(_Q)]

        init = tuple(jnp.full((_L,), 4.0 * _BIG, jnp.float32)
                     for _ in range(_Q))

        @plsc.parallel_loop(0, _NJ, unroll=_UNROLL, carry=init)
        def rmins(j, carry):
            rr = list(carry)
            o = j * _L
            kxv = kx[pl.ds(o, _L)]
            kyv = ky[pl.ds(o, _L)]
            kzv = kz[pl.ds(o, _L)]
            ksv = ks[pl.ds(o, _L)]
            ts = []
            for q in range(_Q):
                t = (ksv + ax[q] * kxv + ay[q] * kyv + az[q] * kzv
                     + sv[q])
                rr[q] = jnp.minimum(rr[q], t)
                ts.append(t)
            while len(ts) > 1:
                ts = [jnp.minimum(ts[2 * i], ts[2 * i + 1])
                      for i in range(len(ts) // 2)]
            cmin = ts[0]
            if first:
                vcm[pl.ds(o, _L)] = cmin
            else:
                vcm[pl.ds(o, _L)] = jnp.minimum(vcm[pl.ds(o, _L)], cmin)
            return tuple(rr)
        for q in range(_Q):
            acc = acc + splat(cqm, q) * jnp.min(rmins[q])
        return acc
      return qb_body

    acc = make_qb_body(True)(0, jnp.zeros((_L,), jnp.float32))
    acc = lax.fori_loop(1, _ROWS // _Q, make_qb_body(False), acc)

    vout[...] = acc
    pltpu.sync_copy(vout, o12h.at[wid])
    pltpu.sync_copy(vcm, ocmh.at[wid])


def _cd_body(*refs):
    wid = lax.axis_index("s") * _NC + lax.axis_index("c")
    _cd_impl(wid, *refs)


_cd = pl.kernel(
    _cd_body,
    out_type=[jax.ShapeDtypeStruct((_NW, _L), jnp.float32),
              jax.ShapeDtypeStruct((_NW, _N), jnp.float32)],
    mesh=plsc.VectorSubcoreMesh(core_axis_name="c", subcore_axis_name="s",
                                num_cores=_NC, num_subcores=_NS),
    scratch_types=[pltpu.VMEM((_N,), jnp.float32)] * 4
                  + [pltpu.VMEM((_ROWS,), jnp.float32)] * 5
                  + [pltpu.VMEM((_N,), jnp.float32),
                     pltpu.VMEM((_L,), jnp.float32),
                     pltpu.SemaphoreType.DMA],
    compiler_params=pltpu.CompilerParams(needs_layout_passes=False),
)


def _epi_body(o12_ref, ocm_ref, mf_ref, out_ref):
    cmn = jnp.min(ocm_ref[...], axis=0, keepdims=True)
    mf = mf_ref[...]
    s21 = jnp.sum(mf * cmn)
    s12 = jnp.sum(o12_ref[:, 0:1])
    nv = jnp.maximum(jnp.sum(mf), 1.0)
    out_ref[...] = jnp.full((1, 1), (s12 + s21) / nv, jnp.float32)


_epi = pl.pallas_call(
    _epi_body,
    out_shape=jax.ShapeDtypeStruct((1, 1), jnp.float32),
)


@jax.jit
def kernel(pred, target):
    outs = _prep(target[0, 0], pred[0, 0])
    flats = [a.reshape(_N) for a in outs]
    mf2d = outs[8].reshape(1, _N)
    o12, ocm = _cd(*flats)
    return _epi(o12, ocm, mf2d)[0, 0]


# speedup vs baseline: 1.1382x; 1.1382x over previous
import functools

import jax
import jax.numpy as jnp
from jax import lax
from jax.experimental import pallas as pl
from jax.experimental.pallas import tpu as pltpu
from jax.experimental.pallas import tpu_sc as plsc

_CU = 609.5593
_CV = 172.854
_FU = 721.5377
_FV = 721.5377
_P03 = 44.85728
_P13 = 0.2163791
_P23 = 0.002745884
_BIG = 1e10

_H = 64
_W = 64
_N = _H * _W
_L = 16
_NC = 2
_NS = 16
_NW = _NC * _NS
_ROWS = _N // _NW
_NJ = _N // _L
_Q = 4
_UNROLL = 2


def _prep_body(t_ref, p_ref, x1, y1, z1, s1, x2, y2, z2, s2, mf):
    t = t_ref[...]
    p = p_ref[...]
    px = lax.broadcasted_iota(jnp.int32, (_H, _W), 1).astype(jnp.float32)
    py = lax.broadcasted_iota(jnp.int32, (_H, _W), 0).astype(jnp.float32)
    m = t > 0.0
    pen = jnp.where(m, 0.0, _BIG)

    def xyz(d):
        x = (px * (d + _P23) - (_CU * d + _P03)) / _FU
        y = (py * (d + _P23) - (_CV * d + _P13)) / _FV
        return x, y, d

    def rb(v):
        return v.astype(jnp.bfloat16).astype(jnp.float32)

    a, b, c = xyz(t)
    x1[...] = rb(a)
    y1[...] = rb(b)
    z1[...] = rb(c)
    s1[...] = a * a + b * b + c * c + pen
    a, b, c = xyz(p)
    x2[...] = rb(a)
    y2[...] = rb(b)
    z2[...] = rb(c)
    s2[...] = a * a + b * b + c * c + pen
    mf[...] = m.astype(jnp.float32)


_prep = pl.pallas_call(
    _prep_body,
    out_shape=[jax.ShapeDtypeStruct((_H, _W), jnp.float32)] * 9,
)


_DNUMS = lax.GatherDimensionNumbers(
    offset_dims=(), collapsed_slice_dims=(0,), start_index_map=(0,))


def _cd_impl(wid, x1h, y1h, z1h, s1h, x2h, y2h, z2h, s2h, mfh,
             o12h, ocmh,
             kx, ky, kz, ks, qx, qy, qz, qs, qm, vcm, vout, sem):
    base = wid * _ROWS

    copies = [
        pltpu.make_async_copy(x2h, kx, sem),
        pltpu.make_async_copy(y2h, ky, sem),
        pltpu.make_async_copy(z2h, kz, sem),
        pltpu.make_async_copy(s2h, ks, sem),
        pltpu.make_async_copy(x1h.at[pl.ds(base, _ROWS)], qx, sem),
        pltpu.make_async_copy(y1h.at[pl.ds(base, _ROWS)], qy, sem),
        pltpu.make_async_copy(z1h.at[pl.ds(base, _ROWS)], qz, sem),
        pltpu.make_async_copy(s1h.at[pl.ds(base, _ROWS)], qs, sem),
        pltpu.make_async_copy(mfh.at[pl.ds(base, _ROWS)], qm, sem),
    ]
    for c in copies:
        c.start()
    for c in copies:
        c.wait()

    def make_qb_body(first):
      def qb_body(qb, acc):
        q0 = qb * _Q
        c0 = (q0 // _L) * _L
        cqx = qx[pl.ds(c0, _L)]
        cqy = qy[pl.ds(c0, _L)]
        cqz = qz[pl.ds(c0, _L)]
        cqs = qs[pl.ds(c0, _L)]
        cqm = qm[pl.ds(c0, _L)]

        def splat(chunk, u):
            lanes = jnp.full((_L,), q0 - c0 + u, jnp.int32)
            return lax.gather(chunk, lanes[:, None], _DNUMS,
                              slice_sizes=(1,),
                              mode=lax.GatherScatterMode.PROMISE_IN_BOUNDS)

        ax = [splat(cqx, u) * -2.0 for u in range(_Q)]
        ay = [splat(cqy, u) * -2.0 for u in range(_Q)]
        az = [splat(cqz, u) * -2.0 for u in range(_Q)]
        sv = [splat(cqs, u) for u in range(_Q)]

        init = tuple(jnp.full((_L,), 4.0 * _BIG, jnp.float32)
                     for _ in range(_Q))

        @plsc.parallel_loop(0, _NJ, unroll=_UNROLL, carry=init)
        def rmins(j, carry):
            rr = list(carry)
            o = j * _L
            kxv = kx[pl.ds(o, _L)]
            kyv = ky[pl.ds(o, _L)]
            kzv = kz[pl.ds(o, _L)]
            ksv = ks[pl.ds(o, _L)]
            ts = []
            for q in range(_Q):
                t = (ksv + ax[q] * kxv + ay[q] * kyv + az[q] * kzv
                     + sv[q])
                rr[q] = jnp.minimum(rr[q], t)
                ts.append(t)
            while len(ts) > 1:
                ts = [jnp.minimum(ts[2 * i], ts[2 * i + 1])
                      for i in range(len(ts) // 2)]
            cmin = ts[0]
            if first:
                vcm[pl.ds(o, _L)] = cmin
            else:
                vcm[pl.ds(o, _L)] = jnp.minimum(vcm[pl.ds(o, _L)], cmin)
            return tuple(rr)
        for q in range(_Q):
            acc = acc + splat(cqm, q) * jnp.min(rmins[q])
        return acc
      return qb_body

    acc = make_qb_body(True)(0, jnp.zeros((_L,), jnp.float32))
    acc = lax.fori_loop(1, _ROWS // _Q, make_qb_body(False), acc)

    vout[...] = acc
    pltpu.sync_copy(vout, o12h.at[wid])
    pltpu.sync_copy(vcm, ocmh.at[wid])


def _cd_body(*refs):
    wid = lax.axis_index("s") * _NC + lax.axis_index("c")
    _cd_impl(wid, *refs)


_cd = pl.kernel(
    _cd_body,
    out_type=[jax.ShapeDtypeStruct((_NW, _L), jnp.float32),
              jax.ShapeDtypeStruct((_NW, _N), jnp.float32)],
    mesh=plsc.VectorSubcoreMesh(core_axis_name="c", subcore_axis_name="s",
                                num_cores=_NC, num_subcores=_NS),
    scratch_types=[pltpu.VMEM((_N,), jnp.float32)] * 4
                  + [pltpu.VMEM((_ROWS,), jnp.float32)] * 5
                  + [pltpu.VMEM((_N,), jnp.float32),
                     pltpu.VMEM((_L,), jnp.float32),
                     pltpu.SemaphoreType.DMA],
    compiler_params=pltpu.CompilerParams(needs_layout_passes=False),
)


def _epi_body(o12_ref, ocm_ref, mf_ref, out_ref):
    cmn = jnp.min(ocm_ref[...], axis=0, keepdims=True)
    mf = mf_ref[...]
    s21 = jnp.sum(mf * cmn)
    s12 = jnp.sum(o12_ref[:, 0:1])
    nv = jnp.maximum(jnp.sum(mf), 1.0)
    out_ref[...] = jnp.full((1, 1), (s12 + s21) / nv, jnp.float32)


_epi = pl.pallas_call(
    _epi_body,
    out_shape=jax.ShapeDtypeStruct((1, 1), jnp.float32),
)


@jax.jit
def kernel(pred, target):
    outs = _prep(target[0, 0], pred[0, 0])
    flats = [a.reshape(_N) for a in outs]
    mf2d = outs[8].reshape(1, _N)
    o12, ocm = _cd(*flats)
    return _epi(o12, ocm, mf2d)[0, 0]
